# baseline (speedup 1.0000x reference)
import functools

import jax
import jax.numpy as jnp
import numpy as _np
from jax import lax
from jax.experimental import pallas as pl
from jax.experimental.pallas import tpu as pltpu
from jax.experimental.pallas import tpu_sc as plsc

N = 10000
E = 320000
D_IN = 128
D = 256
Q = 4
DQ = D // Q
L = 16
NC = 2
NS = 16
NW = NC * NS
NPAD = 10240
BN = 2560
K = 128
S_TOT = 164
S_C0 = 62
S_C1 = S_TOT - S_C0
S_MAX = max(S_C0, S_C1)
E_PAD = NS * S_TOT * K
ROWS_PT = NPAD // NS


def _k1_body(x_ref, w_ref, aw_ref, h3_ref, av_ref):
    h = jnp.dot(x_ref[...], w_ref[...], preferred_element_type=jnp.float32)
    hb = h.astype(jnp.bfloat16)
    for q in range(Q):
        h3_ref[q] = hb[:, q * DQ:(q + 1) * DQ]
    av_ref[...] = jnp.dot(h, aw_ref[...], preferred_element_type=jnp.float32)


def _stage_k1(x_pad, W1, aw1):
    return pl.pallas_call(
        _k1_body,
        grid=(NPAD // BN,),
        in_specs=[
            pl.BlockSpec((BN, D_IN), lambda i: (i, 0)),
            pl.BlockSpec((D_IN, D), lambda i: (0, 0)),
            pl.BlockSpec((D, 128), lambda i: (0, 0)),
        ],
        out_specs=[
            pl.BlockSpec((Q, BN, DQ), lambda i: (0, i, 0)),
            pl.BlockSpec((BN, 128), lambda i: (i, 0)),
        ],
        out_shape=[
            jax.ShapeDtypeStruct((Q, NPAD, DQ), jnp.bfloat16),
            jax.ShapeDtypeStruct((NPAD, 128), jnp.float32),
        ],
    )(x_pad, W1, aw1)


def _sc_body(h3, srch, dsth, ash, adh, outp, denp,
             src_v, dst_v, ex_v, as_v, ad_v, rb0, rb1, sb0, sb1, zbuf,
             accum, dn, g0, g1, s0, s1, dsem):
    c = lax.axis_index("c")
    s = lax.axis_index("s")
    wid = s * NC + c
    n_st = jnp.where(c == 0, S_C0, S_C1)
    rb = (rb0, rb1)
    sb = (sb0, sb1)
    gsem = (g0, g1)
    ssem = (s0, s1)
    pltpu.sync_copy(srch.at[s, pl.ds(c * S_C0, S_MAX)], src_v)
    pltpu.sync_copy(dsth.at[s, pl.ds(c * S_C0, S_MAX)], dst_v)
    pltpu.sync_copy(ash, as_v)
    pltpu.sync_copy(adh, ad_v)
    z16 = jnp.zeros((L,), jnp.float32)

    def zzb(i, _):
        zbuf[pl.ds(i * L, L)] = z16
        return ()
    lax.fori_loop(0, ROWS_PT // L, zzb, ())
    pltpu.sync_copy(zbuf, dn.at[pl.ds(s * ROWS_PT, ROWS_PT)])

    @plsc.parallel_loop(0, S_MAX)
    def estep(st):
        for i in range(K // L):
            sl = pl.ds(i * L, L)
            s16 = src_v[st, sl]
            d16 = dst_v[st, sl]
            va = plsc.load_gather(as_v, [s16])
            vb = plsc.load_gather(ad_v, [d16])
            v = va + vb
            v = jnp.maximum(v, 0.2 * v)
            ex_v[st, sl] = jnp.exp(v)
    plsc.subcore_barrier()

    def dstep(st, _):
        pltpu.async_copy(ex_v.at[st], dn.at[dst_v.at[st]], dsem, add=True)
        return ()
    lax.fori_loop(0, n_st, dstep, ())

    def ddrain(st, _):
        pltpu.make_async_copy(ex_v.at[st], dn.at[dst_v.at[st]], dsem).wait()
        return ()
    lax.fori_loop(0, n_st, ddrain, ())
    plsc.subcore_barrier()
    pltpu.sync_copy(dn.at[pl.ds(s * ROWS_PT, ROWS_PT)],
                    denp.at[c, pl.ds(s * ROWS_PT, ROWS_PT)])

    def zrows(i, _):
        for d in range(DQ // L):
            sb0[i, pl.ds(d * L, L)] = z16
        return ()

    def quarter(q, _):
        hsrc = h3.at[q]
        lax.fori_loop(0, K, zrows, ())
        for j in range(ROWS_PT // K):
            pltpu.sync_copy(sb0, accum.at[pl.ds(s * ROWS_PT + j * K, K)])
        plsc.subcore_barrier()

        def gstart(st, b):
            pltpu.async_copy(hsrc.at[src_v.at[st]], rb[b], gsem[b])

        def gwait(st, b):
            pltpu.make_async_copy(hsrc.at[src_v.at[st]], rb[b],
                                  gsem[b]).wait()

        def sstart(st, b):
            pltpu.async_copy(sb[b], accum.at[dst_v.at[st]], ssem[b],
                             add=True)

        def swait(st, b):
            pltpu.make_async_copy(sb[b], accum.at[dst_v.at[st]],
                                  ssem[b]).wait()

        def scale(st, b):
            @plsc.parallel_loop(0, K // L, unroll=2)
            def scl(i):
                exv = ex_v[st, pl.ds(i * L, L)]
                for j in range(L):
                    a = exv[j]
                    row = i * L + j
                    for d2 in range(DQ // 32):
                        vbf = rb[b][row, pl.ds(d2 * 32, 32)]
                        u, v = plsc.unpack(
                            vbf, format=plsc.PackFormat.INTERLEAVED)
                        sb[b][row, pl.ds(d2 * 32, L)] = u * a
                        sb[b][row, pl.ds(d2 * 32 + L, L)] = v * a

        for b in range(2):
            gstart(b, b)
        for b in range(2):
            gwait(b, b)
            scale(b, b)
            gstart(b + 2, b)
            sstart(b, b)

        def hstep(i, _):
            for b in range(2):
                st = 2 * i + b
                gwait(st, b)
                swait(st - 2, b)
                scale(st, b)
                gstart(st + 2, b)
                sstart(st, b)
            return ()
        lax.fori_loop(1, n_st // 2 - 1, hstep, ())

        for b in range(2):
            st = n_st - 2 + b
            gwait(st, b)
            swait(st - 2, b)
            scale(st, b)
            sstart(st, b)
        for b in range(2):
            swait(n_st - 2 + b, b)

        plsc.subcore_barrier()
        for j in range(ROWS_PT // K):
            sl = pl.ds(s * ROWS_PT + j * K, K)
            pltpu.sync_copy(accum.at[sl], outp.at[c, q, sl])
        plsc.subcore_barrier()
        return ()

    lax.fori_loop(0, Q, quarter, ())


@functools.cache
def _sc_edge_kernel():
    mesh = plsc.VectorSubcoreMesh(core_axis_name="c", subcore_axis_name="s")
    return functools.partial(
        pl.kernel,
        mesh=mesh,
        compiler_params=pltpu.CompilerParams(
            needs_layout_passes=False, use_tc_tiling_on_sc=False),
        out_type=[
            jax.ShapeDtypeStruct((NC, Q, NPAD, DQ), jnp.float32),
            jax.ShapeDtypeStruct((NC, NPAD), jnp.float32),
        ],
        scratch_types=[
            pltpu.VMEM((S_MAX, K), jnp.int32),
            pltpu.VMEM((S_MAX, K), jnp.int32),
            pltpu.VMEM((S_MAX, K), jnp.float32),
            pltpu.VMEM((NPAD,), jnp.float32),
            pltpu.VMEM((NPAD,), jnp.float32),
            pltpu.VMEM((K, DQ), jnp.bfloat16),
            pltpu.VMEM((K, DQ), jnp.bfloat16),
            pltpu.VMEM((K, DQ), jnp.float32),
            pltpu.VMEM((K, DQ), jnp.float32),
            pltpu.VMEM((ROWS_PT,), jnp.float32),
            pltpu.VMEM_SHARED((NPAD, DQ), jnp.float32),
            pltpu.VMEM_SHARED((NPAD,), jnp.float32),
            pltpu.SemaphoreType.DMA,
            pltpu.SemaphoreType.DMA,
            pltpu.SemaphoreType.DMA,
            pltpu.SemaphoreType.DMA,
            pltpu.SemaphoreType.DMA,
        ],
    )(_sc_body)


def _sc_edge(h3, src3, dst3, asv, adv):
    return _sc_edge_kernel()(h3, src3, dst3, asv, adv)


def _e1_body(p_ref, d_ref, b_ref, w_ref, aw_ref,
             x1_ref, h3_ref, av_ref):
    den = jnp.sum(d_ref[...], axis=0) + 1e-16
    p = p_ref[...]
    num = jnp.concatenate([p[0, i] + p[1, i] for i in range(Q)], axis=1)
    x1 = jnp.maximum(num / den[:, None] + b_ref[...], 0.0)
    x1_ref[...] = x1
    h = jnp.dot(x1, w_ref[...], preferred_element_type=jnp.float32)
    hb = h.astype(jnp.bfloat16)
    for q in range(Q):
        h3_ref[q] = hb[:, q * DQ:(q + 1) * DQ]
    av_ref[...] = jnp.dot(h, aw_ref[...], preferred_element_type=jnp.float32)


def _stage_e1(outp, denp, b_row, W2, aw2):
    return pl.pallas_call(
        _e1_body,
        grid=(NPAD // BN,),
        in_specs=[
            pl.BlockSpec((NC, Q, BN, DQ), lambda i: (0, 0, i, 0)),
            pl.BlockSpec((NC, BN), lambda i: (0, i)),
            pl.BlockSpec((1, D), lambda i: (0, 0)),
            pl.BlockSpec((D, D), lambda i: (0, 0)),
            pl.BlockSpec((D, 128), lambda i: (0, 0)),
        ],
        out_specs=[
            pl.BlockSpec((BN, D), lambda i: (i, 0)),
            pl.BlockSpec((Q, BN, DQ), lambda i: (0, i, 0)),
            pl.BlockSpec((BN, 128), lambda i: (i, 0)),
        ],
        out_shape=[
            jax.ShapeDtypeStruct((NPAD, D), jnp.float32),
            jax.ShapeDtypeStruct((Q, NPAD, DQ), jnp.bfloat16),
            jax.ShapeDtypeStruct((NPAD, 128), jnp.float32),
        ],
    )(outp, denp, b_row, W2, aw2)


def _e2_body(p_ref, d_ref, b_ref, x2_ref):
    den = jnp.sum(d_ref[...], axis=0) + 1e-16
    p = p_ref[...]
    num = jnp.concatenate([p[0, i] + p[1, i] for i in range(Q)], axis=1)
    x2_ref[...] = jnp.maximum(num / den[:, None] + b_ref[...], 0.0)


def _stage_e2(outp, denp, b_row):
    return pl.pallas_call(
        _e2_body,
        grid=(NPAD // BN,),
        in_specs=[
            pl.BlockSpec((NC, Q, BN, DQ), lambda i: (0, 0, i, 0)),
            pl.BlockSpec((NC, BN), lambda i: (0, i)),
            pl.BlockSpec((1, D), lambda i: (0, 0)),
        ],
        out_specs=[pl.BlockSpec((BN, D), lambda i: (i, 0))],
        out_shape=[jax.ShapeDtypeStruct((NPAD, D), jnp.float32)],
    )(outp, denp, b_row)


def kernel(x, edge_index, W1, a_src1, a_dst1, b1, W2, a_src2, a_dst2, b2):
    loops = jnp.arange(N, dtype=jnp.int32)
    src = jnp.concatenate([edge_index[0].astype(jnp.int32), loops])
    dst = jnp.concatenate([edge_index[1].astype(jnp.int32), loops])
    pad_e = E_PAD - (E + N)
    src3 = jnp.concatenate([src, jnp.zeros((pad_e,), jnp.int32)])
    dst3 = jnp.concatenate([dst, jnp.full((pad_e,), N, jnp.int32)])
    src3 = src3.reshape(NS, S_TOT, K)
    dst3 = dst3.reshape(NS, S_TOT, K)
    x_pad = jnp.pad(x, ((0, NPAD - N), (0, 0)))
    q32, r32 = _np.arange(D) // 32, _np.arange(D) % 32
    perm = 32 * q32 + _np.where(r32 % 2 == 0, r32 // 2, 16 + (r32 - 1) // 2)
    aw1 = jnp.zeros((D, 128), jnp.float32).at[:, 0].set(a_src1).at[:, 1].set(a_dst1)
    aw2 = jnp.zeros((D, 128), jnp.float32).at[:, 0].set(a_src2).at[:, 1].set(a_dst2)
    W1p, aw1p = W1[:, perm], aw1[perm]
    W2p, aw2p = W2[:, perm], aw2[perm]

    h31, av1 = _stage_k1(x_pad, W1p, aw1p)
    outp1, denp1 = _sc_edge(h31, src3, dst3, av1[:, 0], av1[:, 1])
    x1, h32, av2 = _stage_e1(outp1, denp1, b1.reshape(1, D), W2p, aw2p)
    outp2, denp2 = _sc_edge(h32, src3, dst3, av2[:, 0], av2[:, 1])
    x2 = _stage_e2(outp2, denp2, b2.reshape(1, D))[0]
    return jnp.concatenate([x1[:N], x2[:N]], axis=1)


# speedup vs baseline: 1.1066x; 1.1066x over previous
import functools

import jax
import jax.numpy as jnp
import numpy as _np
from jax import lax
from jax.experimental import pallas as pl
from jax.experimental.pallas import tpu as pltpu
from jax.experimental.pallas import tpu_sc as plsc

N = 10000
E = 320000
D_IN = 128
D = 256
Q = 4
DQ = D // Q
L = 16
NC = 2
NS = 16
NW = NC * NS
NPAD = 10240
BN = 2560
K = 128
S_TOT = 164
S_C0 = 102
S_C1 = S_TOT - S_C0
S_MAX = max(S_C0, S_C1)
E_PAD = NS * S_TOT * K
S_PAD = S_C0 + S_MAX
ROWS_PT = NPAD // NS


def _k1_body(x_ref, w_ref, aw_ref, h3_ref, av_ref):
    h = jnp.dot(x_ref[...], w_ref[...], preferred_element_type=jnp.float32)
    hb = h.astype(jnp.bfloat16)
    for q in range(Q):
        h3_ref[q] = hb[:, q * DQ:(q + 1) * DQ]
    av_ref[...] = jnp.dot(h, aw_ref[...], preferred_element_type=jnp.float32)


def _stage_k1(x_pad, W1, aw1):
    return pl.pallas_call(
        _k1_body,
        grid=(NPAD // BN,),
        in_specs=[
            pl.BlockSpec((BN, D_IN), lambda i: (i, 0)),
            pl.BlockSpec((D_IN, D), lambda i: (0, 0)),
            pl.BlockSpec((D, 128), lambda i: (0, 0)),
        ],
        out_specs=[
            pl.BlockSpec((Q, BN, DQ), lambda i: (0, i, 0)),
            pl.BlockSpec((BN, 128), lambda i: (i, 0)),
        ],
        out_shape=[
            jax.ShapeDtypeStruct((Q, NPAD, DQ), jnp.bfloat16),
            jax.ShapeDtypeStruct((NPAD, 128), jnp.float32),
        ],
    )(x_pad, W1, aw1)


def _sc_body(h3, srch, dsth, ash, adh, outp, denp,
             src_v, dst_v, ex_v, as_v, ad_v, rb0, rb1, sb0, sb1, zbuf,
             accum, dn, g0, g1, s0, s1, dsem):
    c = lax.axis_index("c")
    s = lax.axis_index("s")
    wid = s * NC + c
    n_st = jnp.where(c == 0, S_C0, S_C1)
    rb = (rb0, rb1)
    sb = (sb0, sb1)
    gsem = (g0, g1)
    ssem = (s0, s1)
    pltpu.sync_copy(srch.at[s, pl.ds(c * S_C0, S_MAX)], src_v)
    pltpu.sync_copy(dsth.at[s, pl.ds(c * S_C0, S_MAX)], dst_v)
    pltpu.sync_copy(ash, as_v)
    pltpu.sync_copy(adh, ad_v)
    z16 = jnp.zeros((L,), jnp.float32)

    def zzb(i, _):
        zbuf[pl.ds(i * L, L)] = z16
        return ()
    lax.fori_loop(0, ROWS_PT // L, zzb, ())
    pltpu.sync_copy(zbuf, dn.at[pl.ds(s * ROWS_PT, ROWS_PT)])

    @plsc.parallel_loop(0, S_MAX)
    def estep(st):
        for i in range(K // L):
            sl = pl.ds(i * L, L)
            s16 = src_v[st, sl]
            d16 = dst_v[st, sl]
            va = plsc.load_gather(as_v, [s16])
            vb = plsc.load_gather(ad_v, [d16])
            v = va + vb
            v = jnp.maximum(v, 0.2 * v)
            ex_v[st, sl] = jnp.exp(v)
    plsc.subcore_barrier()

    def dstep(st, _):
        pltpu.async_copy(ex_v.at[st], dn.at[dst_v.at[st]], dsem, add=True)
        return ()
    lax.fori_loop(0, n_st, dstep, ())

    def ddrain(st, _):
        pltpu.make_async_copy(ex_v.at[st], dn.at[dst_v.at[st]], dsem).wait()
        return ()
    lax.fori_loop(0, n_st, ddrain, ())
    plsc.subcore_barrier()
    pltpu.sync_copy(dn.at[pl.ds(s * ROWS_PT, ROWS_PT)],
                    denp.at[c, pl.ds(s * ROWS_PT, ROWS_PT)])

    def zrows(i, _):
        for d in range(DQ // L):
            sb0[i, pl.ds(d * L, L)] = z16
        return ()

    def quarter(q, _):
        hsrc = h3.at[q]
        lax.fori_loop(0, K, zrows, ())
        for j in range(ROWS_PT // K):
            pltpu.sync_copy(sb0, accum.at[pl.ds(s * ROWS_PT + j * K, K)])
        plsc.subcore_barrier()

        def gstart(st, b):
            pltpu.async_copy(hsrc.at[src_v.at[st]], rb[b], gsem[b])

        def gwait(st, b):
            pltpu.make_async_copy(hsrc.at[src_v.at[st]], rb[b],
                                  gsem[b]).wait()

        def sstart(st, b):
            pltpu.async_copy(sb[b], accum.at[dst_v.at[st]], ssem[b],
                             add=True)

        def swait(st, b):
            pltpu.make_async_copy(sb[b], accum.at[dst_v.at[st]],
                                  ssem[b]).wait()

        def scale(st, b):
            @plsc.parallel_loop(0, K // L, unroll=2)
            def scl(i):
                exv = ex_v[st, pl.ds(i * L, L)]
                for j in range(L):
                    a = exv[j]
                    row = i * L + j
                    for d2 in range(DQ // 32):
                        vbf = rb[b][row, pl.ds(d2 * 32, 32)]
                        u, v = plsc.unpack(
                            vbf, format=plsc.PackFormat.INTERLEAVED)
                        sb[b][row, pl.ds(d2 * 32, L)] = u * a
                        sb[b][row, pl.ds(d2 * 32 + L, L)] = v * a

        for b in range(2):
            gstart(b, b)
        for b in range(2):
            gwait(b, b)
            scale(b, b)
            gstart(b + 2, b)
            sstart(b, b)

        def hstep(i, _):
            for b in range(2):
                st = 2 * i + b
                gwait(st, b)
                swait(st - 2, b)
                scale(st, b)
                gstart(st + 2, b)
                sstart(st, b)
            return ()
        lax.fori_loop(1, n_st // 2 - 1, hstep, ())

        for b in range(2):
            st = n_st - 2 + b
            gwait(st, b)
            swait(st - 2, b)
            scale(st, b)
            sstart(st, b)
        for b in range(2):
            swait(n_st - 2 + b, b)

        plsc.subcore_barrier()
        for j in range(ROWS_PT // K):
            sl = pl.ds(s * ROWS_PT + j * K, K)
            pltpu.sync_copy(accum.at[sl], outp.at[c, q, sl])
        plsc.subcore_barrier()
        return ()

    lax.fori_loop(0, Q, quarter, ())


@functools.cache
def _sc_edge_kernel():
    mesh = plsc.VectorSubcoreMesh(core_axis_name="c", subcore_axis_name="s")
    return functools.partial(
        pl.kernel,
        mesh=mesh,
        compiler_params=pltpu.CompilerParams(
            needs_layout_passes=False, use_tc_tiling_on_sc=False),
        out_type=[
            jax.ShapeDtypeStruct((NC, Q, NPAD, DQ), jnp.float32),
            jax.ShapeDtypeStruct((NC, NPAD), jnp.float32),
        ],
        scratch_types=[
            pltpu.VMEM((S_MAX, K), jnp.int32),
            pltpu.VMEM((S_MAX, K), jnp.int32),
            pltpu.VMEM((S_MAX, K), jnp.float32),
            pltpu.VMEM((NPAD,), jnp.float32),
            pltpu.VMEM((NPAD,), jnp.float32),
            pltpu.VMEM((K, DQ), jnp.bfloat16),
            pltpu.VMEM((K, DQ), jnp.bfloat16),
            pltpu.VMEM((K, DQ), jnp.float32),
            pltpu.VMEM((K, DQ), jnp.float32),
            pltpu.VMEM((ROWS_PT,), jnp.float32),
            pltpu.VMEM_SHARED((NPAD, DQ), jnp.float32),
            pltpu.VMEM_SHARED((NPAD,), jnp.float32),
            pltpu.SemaphoreType.DMA,
            pltpu.SemaphoreType.DMA,
            pltpu.SemaphoreType.DMA,
            pltpu.SemaphoreType.DMA,
            pltpu.SemaphoreType.DMA,
        ],
    )(_sc_body)


def _sc_edge(h3, src3, dst3, asv, adv):
    return _sc_edge_kernel()(h3, src3, dst3, asv, adv)


def _e1_body(p_ref, d_ref, b_ref, w_ref, aw_ref,
             x1_ref, h3_ref, av_ref):
    den = jnp.sum(d_ref[...], axis=0) + 1e-16
    p = p_ref[...]
    num = jnp.concatenate([p[0, i] + p[1, i] for i in range(Q)], axis=1)
    x1 = jnp.maximum(num / den[:, None] + b_ref[...], 0.0)
    x1_ref[...] = x1
    h = jnp.dot(x1, w_ref[...], preferred_element_type=jnp.float32)
    hb = h.astype(jnp.bfloat16)
    for q in range(Q):
        h3_ref[q] = hb[:, q * DQ:(q + 1) * DQ]
    av_ref[...] = jnp.dot(h, aw_ref[...], preferred_element_type=jnp.float32)


def _stage_e1(outp, denp, b_row, W2, aw2):
    return pl.pallas_call(
        _e1_body,
        grid=(NPAD // BN,),
        in_specs=[
            pl.BlockSpec((NC, Q, BN, DQ), lambda i: (0, 0, i, 0)),
            pl.BlockSpec((NC, BN), lambda i: (0, i)),
            pl.BlockSpec((1, D), lambda i: (0, 0)),
            pl.BlockSpec((D, D), lambda i: (0, 0)),
            pl.BlockSpec((D, 128), lambda i: (0, 0)),
        ],
        out_specs=[
            pl.BlockSpec((BN, D), lambda i: (i, 0)),
            pl.BlockSpec((Q, BN, DQ), lambda i: (0, i, 0)),
            pl.BlockSpec((BN, 128), lambda i: (i, 0)),
        ],
        out_shape=[
            jax.ShapeDtypeStruct((NPAD, D), jnp.float32),
            jax.ShapeDtypeStruct((Q, NPAD, DQ), jnp.bfloat16),
            jax.ShapeDtypeStruct((NPAD, 128), jnp.float32),
        ],
    )(outp, denp, b_row, W2, aw2)


def _e2_body(p_ref, d_ref, b_ref, x2_ref):
    den = jnp.sum(d_ref[...], axis=0) + 1e-16
    p = p_ref[...]
    num = jnp.concatenate([p[0, i] + p[1, i] for i in range(Q)], axis=1)
    x2_ref[...] = jnp.maximum(num / den[:, None] + b_ref[...], 0.0)


def _stage_e2(outp, denp, b_row):
    return pl.pallas_call(
        _e2_body,
        grid=(NPAD // BN,),
        in_specs=[
            pl.BlockSpec((NC, Q, BN, DQ), lambda i: (0, 0, i, 0)),
            pl.BlockSpec((NC, BN), lambda i: (0, i)),
            pl.BlockSpec((1, D), lambda i: (0, 0)),
        ],
        out_specs=[pl.BlockSpec((BN, D), lambda i: (i, 0))],
        out_shape=[jax.ShapeDtypeStruct((NPAD, D), jnp.float32)],
    )(outp, denp, b_row)


def kernel(x, edge_index, W1, a_src1, a_dst1, b1, W2, a_src2, a_dst2, b2):
    loops = jnp.arange(N, dtype=jnp.int32)
    src = jnp.concatenate([edge_index[0].astype(jnp.int32), loops])
    dst = jnp.concatenate([edge_index[1].astype(jnp.int32), loops])
    pad_e = E_PAD - (E + N)
    src3 = jnp.concatenate([src, jnp.zeros((pad_e,), jnp.int32)])
    dst3 = jnp.concatenate([dst, jnp.full((pad_e,), N, jnp.int32)])
    src3 = src3.reshape(NS, S_TOT, K)
    dst3 = dst3.reshape(NS, S_TOT, K)
    tail = S_PAD - S_TOT
    if tail:
        src3 = jnp.concatenate(
            [src3, jnp.zeros((NS, tail, K), jnp.int32)], axis=1)
        dst3 = jnp.concatenate(
            [dst3, jnp.full((NS, tail, K), N, jnp.int32)], axis=1)
    x_pad = jnp.pad(x, ((0, NPAD - N), (0, 0)))
    q32, r32 = _np.arange(D) // 32, _np.arange(D) % 32
    perm = 32 * q32 + _np.where(r32 % 2 == 0, r32 // 2, 16 + (r32 - 1) // 2)
    aw1 = jnp.zeros((D, 128), jnp.float32).at[:, 0].set(a_src1).at[:, 1].set(a_dst1)
    aw2 = jnp.zeros((D, 128), jnp.float32).at[:, 0].set(a_src2).at[:, 1].set(a_dst2)
    W1p, aw1p = W1[:, perm], aw1[perm]
    W2p, aw2p = W2[:, perm], aw2[perm]

    h31, av1 = _stage_k1(x_pad, W1p, aw1p)
    outp1, denp1 = _sc_edge(h31, src3, dst3, av1[:, 0], av1[:, 1])
    x1, h32, av2 = _stage_e1(outp1, denp1, b1.reshape(1, D), W2p, aw2p)
    outp2, denp2 = _sc_edge(h32, src3, dst3, av2[:, 0], av2[:, 1])
    x2 = _stage_e2(outp2, denp2, b2.reshape(1, D))[0]
    return jnp.concatenate([x1[:N], x2[:N]], axis=1)


# speedup vs baseline: 1.1705x; 1.0577x over previous
import functools

import jax
import jax.numpy as jnp
import numpy as _np
from jax import lax
from jax.experimental import pallas as pl
from jax.experimental.pallas import tpu as pltpu
from jax.experimental.pallas import tpu_sc as plsc

N = 10000
E = 320000
D_IN = 128
D = 256
Q = 4
DQ = D // Q
L = 16
NC = 2
NS = 16
NW = NC * NS
NPAD = 10240
BN = 2560
K = 128
S_TOT = 164
S_C0 = 122
S_C1 = S_TOT - S_C0
S_MAX = max(S_C0, S_C1)
E_PAD = NS * S_TOT * K
S_PAD = S_C0 + S_MAX
ROWS_PT = NPAD // NS


def _k1_body(x_ref, w_ref, aw_ref, h3_ref, av_ref):
    h = jnp.dot(x_ref[...], w_ref[...], preferred_element_type=jnp.float32)
    hb = h.astype(jnp.bfloat16)
    for q in range(Q):
        h3_ref[q] = hb[:, q * DQ:(q + 1) * DQ]
    av_ref[...] = jnp.dot(h, aw_ref[...], preferred_element_type=jnp.float32)


def _stage_k1(x_pad, W1, aw1):
    return pl.pallas_call(
        _k1_body,
        grid=(NPAD // BN,),
        in_specs=[
            pl.BlockSpec((BN, D_IN), lambda i: (i, 0)),
            pl.BlockSpec((D_IN, D), lambda i: (0, 0)),
            pl.BlockSpec((D, 128), lambda i: (0, 0)),
        ],
        out_specs=[
            pl.BlockSpec((Q, BN, DQ), lambda i: (0, i, 0)),
            pl.BlockSpec((BN, 128), lambda i: (i, 0)),
        ],
        out_shape=[
            jax.ShapeDtypeStruct((Q, NPAD, DQ), jnp.bfloat16),
            jax.ShapeDtypeStruct((NPAD, 128), jnp.float32),
        ],
    )(x_pad, W1, aw1)


def _sc_body(h3, srch, dsth, abh, outp, denp,
             src_v, dst_v, ex_v, ab_v, rb0, rb1, sb0, sb1, zbuf,
             accum, dn, g0, g1, s0, s1, dsem):
    c = lax.axis_index("c")
    s = lax.axis_index("s")
    wid = s * NC + c
    n_st = jnp.where(c == 0, S_C0, S_C1)
    rb = (rb0, rb1)
    sb = (sb0, sb1)
    gsem = (g0, g1)
    ssem = (s0, s1)
    pltpu.sync_copy(srch.at[s, pl.ds(c * S_C0, S_MAX)], src_v)
    pltpu.sync_copy(dsth.at[s, pl.ds(c * S_C0, S_MAX)], dst_v)
    pltpu.sync_copy(abh, ab_v)
    z16 = jnp.zeros((L,), jnp.float32)

    def zzb(i, _):
        zbuf[pl.ds(i * L, L)] = z16
        return ()
    lax.fori_loop(0, ROWS_PT // L, zzb, ())
    pltpu.sync_copy(zbuf, dn.at[pl.ds(s * ROWS_PT, ROWS_PT)])

    @plsc.parallel_loop(0, S_MAX)
    def estep(st):
        for i in range(K // L):
            sl = pl.ds(i * L, L)
            s16 = src_v[st, sl]
            d16 = dst_v[st, sl]
            ga = plsc.load_gather(ab_v, [s16])
            gb = plsc.load_gather(ab_v, [d16])
            va, _ = plsc.unpack(plsc.bitcast(ga, jnp.bfloat16),
                                format=plsc.PackFormat.INTERLEAVED)
            _, vb = plsc.unpack(plsc.bitcast(gb, jnp.bfloat16),
                                format=plsc.PackFormat.INTERLEAVED)
            v = va + vb
            v = jnp.maximum(v, 0.2 * v)
            ex_v[st, sl] = jnp.exp(v)
    plsc.subcore_barrier()

    def dstep(st, _):
        pltpu.async_copy(ex_v.at[st], dn.at[dst_v.at[st]], dsem, add=True)
        return ()
    lax.fori_loop(0, n_st, dstep, ())

    def ddrain(st, _):
        pltpu.make_async_copy(ex_v.at[st], dn.at[dst_v.at[st]], dsem).wait()
        return ()
    lax.fori_loop(0, n_st, ddrain, ())
    plsc.subcore_barrier()
    pltpu.sync_copy(dn.at[pl.ds(s * ROWS_PT, ROWS_PT)],
                    denp.at[c, pl.ds(s * ROWS_PT, ROWS_PT)])

    def zrows(i, _):
        for d in range(DQ // L):
            sb0[i, pl.ds(d * L, L)] = z16
        return ()

    def quarter(q, _):
        hsrc = h3.at[q]
        lax.fori_loop(0, K, zrows, ())
        for j in range(ROWS_PT // K):
            pltpu.sync_copy(sb0, accum.at[pl.ds(s * ROWS_PT + j * K, K)])
        plsc.subcore_barrier()

        def gstart(st, b):
            pltpu.async_copy(hsrc.at[src_v.at[st]], rb[b], gsem[b])

        def gwait(st, b):
            pltpu.make_async_copy(hsrc.at[src_v.at[st]], rb[b],
                                  gsem[b]).wait()

        def sstart(st, b):
            pltpu.async_copy(sb[b], accum.at[dst_v.at[st]], ssem[b],
                             add=True)

        def swait(st, b):
            pltpu.make_async_copy(sb[b], accum.at[dst_v.at[st]],
                                  ssem[b]).wait()

        def scale(st, b):
            @plsc.parallel_loop(0, K // L, unroll=2)
            def scl(i):
                exv = ex_v[st, pl.ds(i * L, L)]
                for j in range(L):
                    a = exv[j]
                    row = i * L + j
                    for d2 in range(DQ // 32):
                        vbf = rb[b][row, pl.ds(d2 * 32, 32)]
                        u, v = plsc.unpack(
                            vbf, format=plsc.PackFormat.INTERLEAVED)
                        sb[b][row, pl.ds(d2 * 32, L)] = u * a
                        sb[b][row, pl.ds(d2 * 32 + L, L)] = v * a

        for b in range(2):
            gstart(b, b)
        for b in range(2):
            gwait(b, b)
            scale(b, b)
            gstart(b + 2, b)
            sstart(b, b)

        def hstep(i, _):
            for b in range(2):
                st = 2 * i + b
                gwait(st, b)
                swait(st - 2, b)
                scale(st, b)
                gstart(st + 2, b)
                sstart(st, b)
            return ()
        lax.fori_loop(1, n_st // 2 - 1, hstep, ())

        for b in range(2):
            st = n_st - 2 + b
            gwait(st, b)
            swait(st - 2, b)
            scale(st, b)
            sstart(st, b)
        for b in range(2):
            swait(n_st - 2 + b, b)

        plsc.subcore_barrier()
        for j in range(ROWS_PT // K):
            sl = pl.ds(s * ROWS_PT + j * K, K)
            pltpu.sync_copy(accum.at[sl], outp.at[c, q, sl])
        plsc.subcore_barrier()
        return ()

    lax.fori_loop(0, Q, quarter, ())


@functools.cache
def _sc_edge_kernel():
    mesh = plsc.VectorSubcoreMesh(core_axis_name="c", subcore_axis_name="s")
    return functools.partial(
        pl.kernel,
        mesh=mesh,
        compiler_params=pltpu.CompilerParams(
            needs_layout_passes=False, use_tc_tiling_on_sc=False),
        out_type=[
            jax.ShapeDtypeStruct((NC, Q, NPAD, DQ), jnp.float32),
            jax.ShapeDtypeStruct((NC, NPAD), jnp.float32),
        ],
        scratch_types=[
            pltpu.VMEM((S_MAX, K), jnp.int32),
            pltpu.VMEM((S_MAX, K), jnp.int32),
            pltpu.VMEM((S_MAX, K), jnp.float32),
            pltpu.VMEM((NPAD,), jnp.int32),
            pltpu.VMEM((K, DQ), jnp.bfloat16),
            pltpu.VMEM((K, DQ), jnp.bfloat16),
            pltpu.VMEM((K, DQ), jnp.float32),
            pltpu.VMEM((K, DQ), jnp.float32),
            pltpu.VMEM((ROWS_PT,), jnp.float32),
            pltpu.VMEM_SHARED((NPAD, DQ), jnp.float32),
            pltpu.VMEM_SHARED((NPAD,), jnp.float32),
            pltpu.SemaphoreType.DMA,
            pltpu.SemaphoreType.DMA,
            pltpu.SemaphoreType.DMA,
            pltpu.SemaphoreType.DMA,
            pltpu.SemaphoreType.DMA,
        ],
    )(_sc_body)


def _sc_edge(h3, src3, dst3, asv, adv):
    lo = lax.bitcast_convert_type(
        asv.astype(jnp.bfloat16), jnp.uint16).astype(jnp.uint32)
    hi = lax.bitcast_convert_type(
        adv.astype(jnp.bfloat16), jnp.uint16).astype(jnp.uint32)
    ab = lax.bitcast_convert_type(lo | (hi << 16), jnp.int32)
    return _sc_edge_kernel()(h3, src3, dst3, ab)


def _e1_body(p_ref, d_ref, b_ref, w_ref, aw_ref,
             x1_ref, h3_ref, av_ref):
    den = jnp.sum(d_ref[...], axis=0) + 1e-16
    p = p_ref[...]
    num = jnp.concatenate([p[0, i] + p[1, i] for i in range(Q)], axis=1)
    x1 = jnp.maximum(num / den[:, None] + b_ref[...], 0.0)
    x1_ref[...] = x1
    h = jnp.dot(x1, w_ref[...], preferred_element_type=jnp.float32)
    hb = h.astype(jnp.bfloat16)
    for q in range(Q):
        h3_ref[q] = hb[:, q * DQ:(q + 1) * DQ]
    av_ref[...] = jnp.dot(h, aw_ref[...], preferred_element_type=jnp.float32)


def _stage_e1(outp, denp, b_row, W2, aw2):
    return pl.pallas_call(
        _e1_body,
        grid=(NPAD // BN,),
        in_specs=[
            pl.BlockSpec((NC, Q, BN, DQ), lambda i: (0, 0, i, 0)),
            pl.BlockSpec((NC, BN), lambda i: (0, i)),
            pl.BlockSpec((1, D), lambda i: (0, 0)),
            pl.BlockSpec((D, D), lambda i: (0, 0)),
            pl.BlockSpec((D, 128), lambda i: (0, 0)),
        ],
        out_specs=[
            pl.BlockSpec((BN, D), lambda i: (i, 0)),
            pl.BlockSpec((Q, BN, DQ), lambda i: (0, i, 0)),
            pl.BlockSpec((BN, 128), lambda i: (i, 0)),
        ],
        out_shape=[
            jax.ShapeDtypeStruct((NPAD, D), jnp.float32),
            jax.ShapeDtypeStruct((Q, NPAD, DQ), jnp.bfloat16),
            jax.ShapeDtypeStruct((NPAD, 128), jnp.float32),
        ],
    )(outp, denp, b_row, W2, aw2)


def _e2_body(p_ref, d_ref, b_ref, x2_ref):
    den = jnp.sum(d_ref[...], axis=0) + 1e-16
    p = p_ref[...]
    num = jnp.concatenate([p[0, i] + p[1, i] for i in range(Q)], axis=1)
    x2_ref[...] = jnp.maximum(num / den[:, None] + b_ref[...], 0.0)


def _stage_e2(outp, denp, b_row):
    return pl.pallas_call(
        _e2_body,
        grid=(NPAD // BN,),
        in_specs=[
            pl.BlockSpec((NC, Q, BN, DQ), lambda i: (0, 0, i, 0)),
            pl.BlockSpec((NC, BN), lambda i: (0, i)),
            pl.BlockSpec((1, D), lambda i: (0, 0)),
        ],
        out_specs=[pl.BlockSpec((BN, D), lambda i: (i, 0))],
        out_shape=[jax.ShapeDtypeStruct((NPAD, D), jnp.float32)],
    )(outp, denp, b_row)


def kernel(x, edge_index, W1, a_src1, a_dst1, b1, W2, a_src2, a_dst2, b2):
    loops = jnp.arange(N, dtype=jnp.int32)
    src = jnp.concatenate([edge_index[0].astype(jnp.int32), loops])
    dst = jnp.concatenate([edge_index[1].astype(jnp.int32), loops])
    pad_e = E_PAD - (E + N)
    src3 = jnp.concatenate([src, jnp.zeros((pad_e,), jnp.int32)])
    dst3 = jnp.concatenate([dst, jnp.full((pad_e,), N, jnp.int32)])
    src3 = src3.reshape(NS, S_TOT, K)
    dst3 = dst3.reshape(NS, S_TOT, K)
    tail = S_PAD - S_TOT
    if tail:
        src3 = jnp.concatenate(
            [src3, jnp.zeros((NS, tail, K), jnp.int32)], axis=1)
        dst3 = jnp.concatenate(
            [dst3, jnp.full((NS, tail, K), N, jnp.int32)], axis=1)
    x_pad = jnp.pad(x, ((0, NPAD - N), (0, 0)))
    q32, r32 = _np.arange(D) // 32, _np.arange(D) % 32
    perm = 32 * q32 + _np.where(r32 % 2 == 0, r32 // 2, 16 + (r32 - 1) // 2)
    aw1 = jnp.zeros((D, 128), jnp.float32).at[:, 0].set(a_src1).at[:, 1].set(a_dst1)
    aw2 = jnp.zeros((D, 128), jnp.float32).at[:, 0].set(a_src2).at[:, 1].set(a_dst2)
    W1p, aw1p = W1[:, perm], aw1[perm]
    W2p, aw2p = W2[:, perm], aw2[perm]

    h31, av1 = _stage_k1(x_pad, W1p, aw1p)
    outp1, denp1 = _sc_edge(h31, src3, dst3, av1[:, 0], av1[:, 1])
    x1, h32, av2 = _stage_e1(outp1, denp1, b1.reshape(1, D), W2p, aw2p)
    outp2, denp2 = _sc_edge(h32, src3, dst3, av2[:, 0], av2[:, 1])
    x2 = _stage_e2(outp2, denp2, b2.reshape(1, D))[0]
    return jnp.concatenate([x1[:N], x2[:N]], axis=1)
